# Initial kernel scaffold; baseline (speedup 1.0000x reference)
import jax
import jax.numpy as jnp
from jax.experimental import pallas as pl


def kernel(input, coefficients_vect):
    raise NotImplementedError("write your pallas kernel here")


# speedup vs baseline: 167.6171x; 167.6171x over previous
import functools

import jax
import jax.numpy as jnp
import numpy as np
from jax import lax
from jax.experimental import pallas as pl
from jax.experimental.pallas import tpu as pltpu
from jax.experimental.pallas import tpu_sc as plsc

_N, _C, _H, _W = 64, 768, 24, 24
_SIZE = 51
_ROW = _H * _W
_NUM_ROWS = _N * _C
_TOTAL = _NUM_ROWS * _ROW
_TABLE = _C * _SIZE

_NC, _NS, _L = 2, 16, 16
_NW = _NC * _NS
_ROWS_PER_W = _NUM_ROWS // _NW
_CHUNK_ROWS = 24
_CHUNK = _CHUNK_ROWS * _ROW
_NCHUNKS = _ROWS_PER_W // _CHUNK_ROWS
_VREGS_PER_ROW = _ROW // _L

_LO = np.float32(-(0.1 * (_SIZE // 2)))
_HI = np.float32(0.1 * (_SIZE // 2 - 1))
_INV_GRID = np.float32(10.0)
_CENTER = _SIZE // 2


def _body(x_hbm, tab_hbm, out_hbm, tab_v, in_v, out_v):
    wid = lax.axis_index("s") * _NC + lax.axis_index("c")
    pltpu.sync_copy(tab_hbm, tab_v)
    worker_base = wid * (_ROWS_PER_W * _ROW)

    def chunk_body(g, _):
        base_elem = worker_base + g * _CHUNK
        pltpu.sync_copy(x_hbm.at[pl.ds(base_elem, _CHUNK)], in_v)
        c0 = lax.rem(g * _CHUNK_ROWS, _C)

        def row_body(j, _):
            base = (c0 + j) * _SIZE + _CENTER

            def vec_body(v, _):
                off = j * _ROW + v * _L
                xv = in_v[pl.ds(off, _L)]
                xc = jnp.minimum(jnp.maximum(xv, _LO), _HI)
                t = xc * _INV_GRID
                it = t.astype(jnp.int32)
                ft = it.astype(jnp.float32)
                fl = jnp.where(ft > t, ft - 1.0, ft)
                frac = t - fl
                idx = fl.astype(jnp.int32) + base
                g0 = plsc.load_gather(tab_v, [idx])
                g1 = plsc.load_gather(tab_v, [idx + 1])
                out_v[pl.ds(off, _L)] = g0 + frac * (g1 - g0)
                return 0

            lax.fori_loop(0, _VREGS_PER_ROW, vec_body, 0)
            return 0

        lax.fori_loop(0, _CHUNK_ROWS, row_body, 0)
        pltpu.sync_copy(out_v, out_hbm.at[pl.ds(base_elem, _CHUNK)])
        return 0

    lax.fori_loop(0, _NCHUNKS, chunk_body, 0)


_sc_kernel = functools.partial(
    pl.kernel,
    out_type=jax.ShapeDtypeStruct((_TOTAL,), jnp.float32),
    mesh=plsc.VectorSubcoreMesh(
        core_axis_name="c", subcore_axis_name="s",
        num_cores=_NC, num_subcores=_NS,
    ),
    scratch_types=[
        pltpu.VMEM((_TABLE,), jnp.float32),
        pltpu.VMEM((_CHUNK,), jnp.float32),
        pltpu.VMEM((_CHUNK,), jnp.float32),
    ],
    compiler_params=pltpu.CompilerParams(needs_layout_passes=False),
)(_body)


@jax.jit
def kernel(input, coefficients_vect):
    out = _sc_kernel(input.reshape(_TOTAL), coefficients_vect)
    return out.reshape(input.shape)


# speedup vs baseline: 171.5847x; 1.0237x over previous
import functools

import jax
import jax.numpy as jnp
import numpy as np
from jax import lax
from jax.experimental import pallas as pl
from jax.experimental.pallas import tpu as pltpu
from jax.experimental.pallas import tpu_sc as plsc

_N, _C, _H, _W = 64, 768, 24, 24
_SIZE = 51
_ROW = _H * _W
_NUM_ROWS = _N * _C
_TOTAL = _NUM_ROWS * _ROW
_TABLE = _C * _SIZE

_NC, _NS, _L = 2, 16, 16
_NW = _NC * _NS
_ROWS_PER_W = _NUM_ROWS // _NW
_CHUNK_ROWS = 24
_CHUNK = _CHUNK_ROWS * _ROW
_NCHUNKS = _ROWS_PER_W // _CHUNK_ROWS
_VREGS_PER_ROW = _ROW // _L

_LO = np.float32(-(0.1 * (_SIZE // 2)))
_HI = np.float32(0.1 * (_SIZE // 2 - 1))
_INV_GRID = np.float32(10.0)
_CENTER = _SIZE // 2


def _body(x_hbm, tab_hbm, out_hbm, tab_v, in_v, out_v):
    wid = lax.axis_index("s") * _NC + lax.axis_index("c")
    pltpu.sync_copy(tab_hbm, tab_v)
    worker_base = wid * (_ROWS_PER_W * _ROW)

    def chunk_body(g, _):
        base_elem = worker_base + g * _CHUNK
        pltpu.sync_copy(x_hbm.at[pl.ds(base_elem, _CHUNK)], in_v)
        c0 = lax.rem(g * _CHUNK_ROWS, _C)

        def row_body(j, _):
            base = (c0 + j) * _SIZE + _CENTER
            row_off = j * _ROW

            for v in range(_VREGS_PER_ROW):
                off = row_off + v * _L
                xv = in_v[pl.ds(off, _L)]
                xc = jnp.minimum(jnp.maximum(xv, _LO), _HI)
                t = xc * _INV_GRID
                it = t.astype(jnp.int32)
                ft = it.astype(jnp.float32)
                fl = jnp.where(ft > t, ft - 1.0, ft)
                frac = t - fl
                idx = fl.astype(jnp.int32) + base
                g0 = plsc.load_gather(tab_v, [idx])
                g1 = plsc.load_gather(tab_v, [idx + 1])
                out_v[pl.ds(off, _L)] = g0 + frac * (g1 - g0)
            return 0

        lax.fori_loop(0, _CHUNK_ROWS, row_body, 0)
        pltpu.sync_copy(out_v, out_hbm.at[pl.ds(base_elem, _CHUNK)])
        return 0

    lax.fori_loop(0, _NCHUNKS, chunk_body, 0)


_sc_kernel = functools.partial(
    pl.kernel,
    out_type=jax.ShapeDtypeStruct((_TOTAL,), jnp.float32),
    mesh=plsc.VectorSubcoreMesh(
        core_axis_name="c", subcore_axis_name="s",
        num_cores=_NC, num_subcores=_NS,
    ),
    scratch_types=[
        pltpu.VMEM((_TABLE,), jnp.float32),
        pltpu.VMEM((_CHUNK,), jnp.float32),
        pltpu.VMEM((_CHUNK,), jnp.float32),
    ],
    compiler_params=pltpu.CompilerParams(needs_layout_passes=False),
)(_body)


@jax.jit
def kernel(input, coefficients_vect):
    out = _sc_kernel(input.reshape(_TOTAL), coefficients_vect)
    return out.reshape(input.shape)


# speedup vs baseline: 177.7886x; 1.0362x over previous
import functools

import jax
import jax.numpy as jnp
import numpy as np
from jax import lax
from jax.experimental import pallas as pl
from jax.experimental.pallas import tpu as pltpu
from jax.experimental.pallas import tpu_sc as plsc

_N, _C, _H, _W = 64, 768, 24, 24
_SIZE = 51
_ROW = _H * _W
_NUM_ROWS = _N * _C
_TOTAL = _NUM_ROWS * _ROW
_TABLE = _C * _SIZE

_NC, _NS, _L = 2, 16, 16
_NW = _NC * _NS
_ROWS_PER_W = _NUM_ROWS // _NW
_CHUNK_ROWS = 32
_CHUNK = _CHUNK_ROWS * _ROW
_NCHUNKS = _ROWS_PER_W // _CHUNK_ROWS
_VREGS_PER_ROW = _ROW // _L

_LO = np.float32(-(0.1 * (_SIZE // 2)))
_HI = np.float32(0.1 * (_SIZE // 2 - 1))
_INV_GRID = np.float32(10.0)
_CENTER = _SIZE // 2


def _body(x_hbm, tab_hbm, out_hbm, tab_v,
          in0, in1, out0, out1, sin0, sin1, sout0, sout1):
    wid = lax.axis_index("s") * _NC + lax.axis_index("c")
    pltpu.sync_copy(tab_hbm, tab_v)
    worker_base = wid * (_ROWS_PER_W * _ROW)

    ins, outs = (in0, in1), (out0, out1)
    sins, souts = (sin0, sin1), (sout0, sout1)

    def src(g):
        return x_hbm.at[pl.ds(worker_base + g * _CHUNK, _CHUNK)]

    def dst(g):
        return out_hbm.at[pl.ds(worker_base + g * _CHUNK, _CHUNK)]

    def compute(g, in_v, out_v):
        c0 = lax.rem(g * _CHUNK_ROWS, _C)

        def row_body(j, _):
            base = (c0 + j) * _SIZE + _CENTER
            row_off = j * _ROW

            for v in range(_VREGS_PER_ROW):
                off = row_off + v * _L
                xv = in_v[pl.ds(off, _L)]
                xc = jnp.minimum(jnp.maximum(xv, _LO), _HI)
                t = xc * _INV_GRID
                it = t.astype(jnp.int32)
                ft = it.astype(jnp.float32)
                fl = jnp.where(ft > t, ft - 1.0, ft)
                frac = t - fl
                idx = fl.astype(jnp.int32) + base
                g0 = plsc.load_gather(tab_v, [idx])
                g1 = plsc.load_gather(tab_v, [idx + 1])
                out_v[pl.ds(off, _L)] = g0 + frac * (g1 - g0)
            return 0

        lax.fori_loop(0, _CHUNK_ROWS, row_body, 0)

    pltpu.make_async_copy(src(0), in0, sin0).start()

    def g2_body(g2, _):
        for b in range(2):
            g = g2 * 2 + b

            @pl.when(g + 1 < _NCHUNKS)
            def _prefetch():
                pltpu.make_async_copy(src(g + 1), ins[1 - b], sins[1 - b]).start()

            pltpu.make_async_copy(src(g), ins[b], sins[b]).wait()

            @pl.when(g >= 2)
            def _drain():
                pltpu.make_async_copy(outs[b], dst(g - 2), souts[b]).wait()

            compute(g, ins[b], outs[b])
            pltpu.make_async_copy(outs[b], dst(g), souts[b]).start()
        return 0

    lax.fori_loop(0, _NCHUNKS // 2, g2_body, 0)
    pltpu.make_async_copy(out0, dst(_NCHUNKS - 2), sout0).wait()
    pltpu.make_async_copy(out1, dst(_NCHUNKS - 1), sout1).wait()


_sc_kernel = functools.partial(
    pl.kernel,
    out_type=jax.ShapeDtypeStruct((_TOTAL,), jnp.float32),
    mesh=plsc.VectorSubcoreMesh(
        core_axis_name="c", subcore_axis_name="s",
        num_cores=_NC, num_subcores=_NS,
    ),
    scratch_types=[
        pltpu.VMEM((_TABLE,), jnp.float32),
        pltpu.VMEM((_CHUNK,), jnp.float32),
        pltpu.VMEM((_CHUNK,), jnp.float32),
        pltpu.VMEM((_CHUNK,), jnp.float32),
        pltpu.VMEM((_CHUNK,), jnp.float32),
        pltpu.SemaphoreType.DMA,
        pltpu.SemaphoreType.DMA,
        pltpu.SemaphoreType.DMA,
        pltpu.SemaphoreType.DMA,
    ],
    compiler_params=pltpu.CompilerParams(needs_layout_passes=False),
)(_body)


@jax.jit
def kernel(input, coefficients_vect):
    out = _sc_kernel(input.reshape(_TOTAL), coefficients_vect)
    return out.reshape(input.shape)


# speedup vs baseline: 277.2539x; 1.5595x over previous
import functools

import jax
import jax.numpy as jnp
import numpy as np
from jax import lax
from jax.experimental import pallas as pl
from jax.experimental.pallas import tpu as pltpu
from jax.experimental.pallas import tpu_sc as plsc

_N, _C, _H, _W = 64, 768, 24, 24
_SIZE = 51
_ROW = _H * _W
_NUM_ROWS = _N * _C
_TOTAL = _NUM_ROWS * _ROW
_TABLE = _C * _SIZE

_NC, _NS, _L = 2, 16, 16
_NW = _NC * _NS
_ROWS_PER_W = _NUM_ROWS // _NW
_CHUNK_ROWS = 32
_CHUNK = _CHUNK_ROWS * _ROW
_NCHUNKS = _ROWS_PER_W // _CHUNK_ROWS
_VREGS_PER_ROW = _ROW // _L

_LO = np.float32(-(0.1 * (_SIZE // 2)))
_HI = np.float32(0.1 * (_SIZE // 2 - 1))
_INV_GRID = np.float32(10.0)
_CENTER = _SIZE // 2
_F_CENTER = np.float32(_CENTER)


def _body(x_hbm, tab_hbm, out_hbm, tab_v,
          in0, in1, out0, out1, sin0, sin1, sout0, sout1):
    wid = lax.axis_index("s") * _NC + lax.axis_index("c")
    pltpu.sync_copy(tab_hbm, tab_v)
    worker_base = wid * (_ROWS_PER_W * _ROW)

    ins, outs = (in0, in1), (out0, out1)
    sins, souts = (sin0, sin1), (sout0, sout1)

    def src(g):
        return x_hbm.at[pl.ds(worker_base + g * _CHUNK, _CHUNK)]

    def dst(g):
        return out_hbm.at[pl.ds(worker_base + g * _CHUNK, _CHUNK)]

    def compute(g, in_v, out_v):
        c0 = lax.rem(g * _CHUNK_ROWS, _C)

        def row_body(j, _):
            base51 = (c0 + j) * _SIZE
            row_off = j * _ROW

            @plsc.parallel_loop(0, _ROW, step=_L, unroll=6)
            def _vec(o):
                off = row_off + o
                xv = in_v[pl.ds(off, _L)]
                xc = jnp.minimum(jnp.maximum(xv, _LO), _HI)
                u = xc * _INV_GRID + _F_CENTER
                iu = u.astype(jnp.int32)
                frac = u - iu.astype(jnp.float32)
                idx = iu + base51
                g0 = plsc.load_gather(tab_v, [idx])
                g1 = plsc.load_gather(tab_v, [idx + 1])
                out_v[pl.ds(off, _L)] = g0 + frac * (g1 - g0)

            return 0

        lax.fori_loop(0, _CHUNK_ROWS, row_body, 0)

    pltpu.make_async_copy(src(0), in0, sin0).start()

    def g2_body(g2, _):
        for b in range(2):
            g = g2 * 2 + b

            @pl.when(g + 1 < _NCHUNKS)
            def _prefetch():
                pltpu.make_async_copy(src(g + 1), ins[1 - b], sins[1 - b]).start()

            pltpu.make_async_copy(src(g), ins[b], sins[b]).wait()

            @pl.when(g >= 2)
            def _drain():
                pltpu.make_async_copy(outs[b], dst(g - 2), souts[b]).wait()

            compute(g, ins[b], outs[b])
            pltpu.make_async_copy(outs[b], dst(g), souts[b]).start()
        return 0

    lax.fori_loop(0, _NCHUNKS // 2, g2_body, 0)
    pltpu.make_async_copy(out0, dst(_NCHUNKS - 2), sout0).wait()
    pltpu.make_async_copy(out1, dst(_NCHUNKS - 1), sout1).wait()


_sc_kernel = functools.partial(
    pl.kernel,
    out_type=jax.ShapeDtypeStruct((_TOTAL,), jnp.float32),
    mesh=plsc.VectorSubcoreMesh(
        core_axis_name="c", subcore_axis_name="s",
        num_cores=_NC, num_subcores=_NS,
    ),
    scratch_types=[
        pltpu.VMEM((_TABLE,), jnp.float32),
        pltpu.VMEM((_CHUNK,), jnp.float32),
        pltpu.VMEM((_CHUNK,), jnp.float32),
        pltpu.VMEM((_CHUNK,), jnp.float32),
        pltpu.VMEM((_CHUNK,), jnp.float32),
        pltpu.SemaphoreType.DMA,
        pltpu.SemaphoreType.DMA,
        pltpu.SemaphoreType.DMA,
        pltpu.SemaphoreType.DMA,
    ],
    compiler_params=pltpu.CompilerParams(needs_layout_passes=False),
)(_body)


@jax.jit
def kernel(input, coefficients_vect):
    out = _sc_kernel(input.reshape(_TOTAL), coefficients_vect)
    return out.reshape(input.shape)


# speedup vs baseline: 2356.9428x; 8.5010x over previous
import functools

import jax
import jax.numpy as jnp
import numpy as np
from jax import lax
from jax.experimental import pallas as pl
from jax.experimental.pallas import tpu as pltpu
from jax.experimental.pallas import tpu_sc as plsc

_N, _C, _H, _W = 64, 768, 24, 24
_SIZE = 51
_TABLE = _C * _SIZE

_NC, _NS, _L = 2, 16, 16
_NW = _NC * _NS
_NSLABS = _N * _H
_SLABS_PER_W = _NSLABS // _NW
_CVECS = _C // _L

_LO = np.float32(-(0.1 * (_SIZE // 2)))
_HI = np.float32(0.1 * (_SIZE // 2 - 1))
_INV_GRID = np.float32(10.0)
_F_CENTER = np.float32(_SIZE // 2)


def _body(x_hbm, tab_hbm, out_hbm, tab_v,
          in0, in1, out0, out1, sin0, sin1, sout0, sout1):
    wid = lax.axis_index("s") * _NC + lax.axis_index("c")
    pltpu.sync_copy(tab_hbm, tab_v)
    slab0 = wid * _SLABS_PER_W

    ins, outs = (in0, in1), (out0, out1)
    sins, souts = (sin0, sin1), (sout0, sout1)

    def src(g):
        return x_hbm.at[slab0 + g]

    def dst(g):
        return out_hbm.at[slab0 + g]

    iota51 = lax.iota(jnp.int32, _L) * _SIZE

    def compute(in_v, out_v):
        def cvec_body(k, _):
            bv = iota51 + k * (_L * _SIZE)
            cb = k * _L

            @plsc.parallel_loop(0, _W, step=1, unroll=6)
            def _row(w):
                xv = in_v[w, pl.ds(cb, _L)]
                xc = jnp.minimum(jnp.maximum(xv, _LO), _HI)
                u = xc * _INV_GRID + _F_CENTER
                iu = u.astype(jnp.int32)
                frac = u - iu.astype(jnp.float32)
                idx = iu + bv
                g0 = plsc.load_gather(tab_v, [idx])
                g1 = plsc.load_gather(tab_v, [idx + 1])
                out_v[w, pl.ds(cb, _L)] = g0 + frac * (g1 - g0)

            return 0

        lax.fori_loop(0, _CVECS, cvec_body, 0)

    pltpu.make_async_copy(src(0), in0, sin0).start()

    def g2_body(g2, _):
        for b in range(2):
            g = g2 * 2 + b

            @pl.when(g + 1 < _SLABS_PER_W)
            def _prefetch():
                pltpu.make_async_copy(src(g + 1), ins[1 - b], sins[1 - b]).start()

            pltpu.make_async_copy(src(g), ins[b], sins[b]).wait()

            @pl.when(g >= 2)
            def _drain():
                pltpu.make_async_copy(outs[b], dst(g - 2), souts[b]).wait()

            compute(ins[b], outs[b])
            pltpu.make_async_copy(outs[b], dst(g), souts[b]).start()
        return 0

    lax.fori_loop(0, _SLABS_PER_W // 2, g2_body, 0)
    pltpu.make_async_copy(out0, dst(_SLABS_PER_W - 2), sout0).wait()
    pltpu.make_async_copy(out1, dst(_SLABS_PER_W - 1), sout1).wait()


_sc_kernel = functools.partial(
    pl.kernel,
    out_type=jax.ShapeDtypeStruct((_NSLABS, _W, _C), jnp.float32),
    mesh=plsc.VectorSubcoreMesh(
        core_axis_name="c", subcore_axis_name="s",
        num_cores=_NC, num_subcores=_NS,
    ),
    scratch_types=[
        pltpu.VMEM((_TABLE,), jnp.float32),
        pltpu.VMEM((_W, _C), jnp.float32),
        pltpu.VMEM((_W, _C), jnp.float32),
        pltpu.VMEM((_W, _C), jnp.float32),
        pltpu.VMEM((_W, _C), jnp.float32),
        pltpu.SemaphoreType.DMA,
        pltpu.SemaphoreType.DMA,
        pltpu.SemaphoreType.DMA,
        pltpu.SemaphoreType.DMA,
    ],
    compiler_params=pltpu.CompilerParams(
        needs_layout_passes=False,
        use_tc_tiling_on_sc=True,
    ),
)(_body)


@jax.jit
def kernel(input, coefficients_vect):
    xp = jnp.transpose(input, (0, 2, 3, 1)).reshape(_NSLABS, _W, _C)
    out = _sc_kernel(xp, coefficients_vect)
    out = out.reshape(_N, _H, _W, _C)
    return jnp.transpose(out, (0, 3, 1, 2))
